# baseline (speedup 1.0000x reference)
import jax
import jax.numpy as jnp
from jax import lax
from jax.experimental import pallas as pl
from jax.experimental.pallas import tpu as pltpu
from jax.experimental.pallas import tpu_sc as plsc

_SIZE = 1000000
_DIM = 64
_MOM = 0.9
_B = 16384

_NC = 2
_NS = 16
_L = 16
_NW = _NC * _NS
_BPW = _B // _NW
_CH = 128
_NCH = _BPW // _CH

_mesh = plsc.VectorSubcoreMesh(core_axis_name="c", subcore_axis_name="s")


def _body(idx_hbm, x_hbm, mem_hbm, newmem_ref, da_out, idx_v, rows_v, x_v, sem):
  wid = lax.axis_index("s") * _NC + lax.axis_index("c")
  base = wid * _BPW
  for j in range(_NCH):
    pltpu.sync_copy(idx_hbm.at[pl.ds(base + j * _CH, _CH)], idx_v.at[j])
  for j in range(_NCH):
    row0 = base + j * _CH
    pltpu.async_copy(mem_hbm.at[idx_v.at[j]], rows_v, sem).wait()
    pltpu.sync_copy(rows_v, da_out.at[pl.ds(row0, _CH)])
    pltpu.sync_copy(x_hbm.at[pl.ds(row0, _CH)], x_v)

    @pl.loop(0, _CH)
    def _(i):
      for k in range(_DIM // _L):
        sl = pl.ds(k * _L, _L)
        x_v[i, sl] = rows_v[i, sl] * _MOM + x_v[i, sl] * (1.0 - _MOM)

    pltpu.async_copy(x_v, newmem_ref.at[idx_v.at[j]], sem).wait()


_sc_call = pl.kernel(
    _body,
    out_type=jax.ShapeDtypeStruct((_B, _DIM), jnp.float32),
    mesh=_mesh,
    scratch_types=[
        pltpu.VMEM((_NCH, _CH), jnp.int32),
        pltpu.VMEM((_CH, _DIM), jnp.float32),
        pltpu.VMEM((_CH, _DIM), jnp.float32),
        pltpu.SemaphoreType.DMA,
    ],
    compiler_params=pltpu.CompilerParams(use_tc_tiling_on_sc=False),
)


@jax.jit
def _run(indices, x, memory):
  newmem = jax.new_ref(memory)
  da = _sc_call(indices, x, memory, newmem)
  return da, jax.freeze(newmem)


def kernel(indices, x, memory):
  return _run(indices, x, memory)


# speedup vs baseline: 2.5733x; 2.5733x over previous
import jax
import jax.numpy as jnp
from jax import lax
from jax.experimental import pallas as pl
from jax.experimental.pallas import tpu as pltpu
from jax.experimental.pallas import tpu_sc as plsc

_SIZE = 1000000
_DIM = 64
_MOM = 0.9
_B = 16384

_NC = 2
_NS = 16
_L = 16
_NW = _NC * _NS
_BPW = _B // _NW
_CH = 128
_NCH = _BPW // _CH

_mesh = plsc.VectorSubcoreMesh(core_axis_name="c", subcore_axis_name="s")


def _gather_body(idx_hbm, x_hbm, mem_ref, da_out, ne_out, idx_v, rows_v, x_v,
                 sem):
  wid = lax.axis_index("s") * _NC + lax.axis_index("c")
  base = wid * _BPW
  for j in range(_NCH):
    pltpu.sync_copy(idx_hbm.at[pl.ds(base + j * _CH, _CH)], idx_v.at[j])
  for j in range(_NCH):
    row0 = base + j * _CH
    pltpu.async_copy(mem_ref.at[idx_v.at[j]], rows_v, sem).wait()
    pltpu.sync_copy(rows_v, da_out.at[pl.ds(row0, _CH)])
    pltpu.sync_copy(x_hbm.at[pl.ds(row0, _CH)], x_v)

    @pl.loop(0, _CH)
    def _(i):
      for k in range(_DIM // _L):
        sl = pl.ds(k * _L, _L)
        x_v[i, sl] = rows_v[i, sl] * _MOM + x_v[i, sl] * (1.0 - _MOM)

    pltpu.sync_copy(x_v, ne_out.at[pl.ds(row0, _CH)])


def _scatter_body(idx_hbm, ne_hbm, mem_ref, idx_v, rows_v, sem):
  wid = lax.axis_index("s") * _NC + lax.axis_index("c")
  base = wid * _BPW
  for j in range(_NCH):
    pltpu.sync_copy(idx_hbm.at[pl.ds(base + j * _CH, _CH)], idx_v.at[j])
  for j in range(_NCH):
    row0 = base + j * _CH
    pltpu.sync_copy(ne_hbm.at[pl.ds(row0, _CH)], rows_v)
    pltpu.async_copy(rows_v, mem_ref.at[idx_v.at[j]], sem).wait()


_gather_call = pl.kernel(
    _gather_body,
    out_type=(
        jax.ShapeDtypeStruct((_B, _DIM), jnp.float32),
        jax.ShapeDtypeStruct((_B, _DIM), jnp.float32),
    ),
    mesh=_mesh,
    scratch_types=[
        pltpu.VMEM((_NCH, _CH), jnp.int32),
        pltpu.VMEM((_CH, _DIM), jnp.float32),
        pltpu.VMEM((_CH, _DIM), jnp.float32),
        pltpu.SemaphoreType.DMA,
    ],
    compiler_params=pltpu.CompilerParams(use_tc_tiling_on_sc=False),
)

_scatter_call = pl.kernel(
    _scatter_body,
    out_type=(),
    mesh=_mesh,
    scratch_types=[
        pltpu.VMEM((_NCH, _CH), jnp.int32),
        pltpu.VMEM((_CH, _DIM), jnp.float32),
        pltpu.SemaphoreType.DMA,
    ],
    compiler_params=pltpu.CompilerParams(use_tc_tiling_on_sc=False),
)


@jax.jit
def _run(indices, x, memory):
  newmem = jax.new_ref(memory)
  da, ne = _gather_call(indices, x, newmem)
  _scatter_call(indices, ne, newmem)
  return da, jax.freeze(newmem)


def kernel(indices, x, memory):
  return _run(indices, x, memory)
